# baseline (speedup 1.0000x reference)
import functools

import jax
import jax.numpy as jnp
from jax import lax
from jax.experimental import pallas as pl
from jax.experimental.pallas import tpu as pltpu
from jax.experimental.pallas import tpu_sc as plsc

_NUM_CORES = 2
_NUM_SUBCORES = 16
_NUM_WORKERS = _NUM_CORES * _NUM_SUBCORES

_KB = 4096
_ROWS = 64
_NEG = -1e30


def _argmax_body(k_total, nkb, xt_ref, kt_ref, out_ref, bm_ref, bb_ref, k2_ref):
    i = pl.program_id(0)
    q = bm_ref.shape[0]

    @pl.when(i == 0)
    def _():
        bm_ref[...] = jnp.full((q, 128), _NEG, jnp.float32)
        bb_ref[...] = jnp.zeros((q, 128), jnp.int32)

    kt = kt_ref[...]
    k2_ref[0:16, :] = kt
    k2_ref[16:17, :] = (-0.5 * jnp.sum(kt * kt, axis=0))[None, :]
    sc = lax.dot_general(
        xt_ref[...], k2_ref[...], (((0,), (0,)), ((), ())),
        preferred_element_type=jnp.float32)

    valid_in_last = k_total - (nkb - 1) * _KB
    nslab = _KB // 128
    base0 = i * _KB

    def scan(scv):
        for qb in range(q // _ROWS):
            row = qb * _ROWS
            m = bm_ref[row:row + _ROWS, :]
            b = bb_ref[row:row + _ROWS, :]
            for j in range(nslab):
                v = scv[row:row + _ROWS, j * 128:(j + 1) * 128]
                upd = v > m
                b = jnp.where(upd, base0 + j * 128, b)
                m = jnp.where(upd, v, m)
            bm_ref[row:row + _ROWS, :] = m
            bb_ref[row:row + _ROWS, :] = b

    @pl.when(i < nkb - 1)
    def _():
        scan(sc)

    @pl.when(i == nkb - 1)
    def _():
        lane = lax.broadcasted_iota(jnp.int32, (q, _KB), 1)
        scan(jnp.where(lane < valid_in_last, sc, _NEG))

    @pl.when(i == nkb - 1)
    def _():
        m = bm_ref[...]
        b = bb_ref[...]
        g = jnp.max(m, axis=1)
        lane = lax.broadcasted_iota(jnp.int32, (q, 128), 1)
        cand = jnp.where(m == g[:, None], b + lane, jnp.int32(0x7FFFFFFF))
        out_ref[...] = jnp.min(cand, axis=1)


def _match_indices(xt, inputs_t):
    d, q = xt.shape
    k_total = inputs_t.shape[1]
    nkb = pl.cdiv(k_total, _KB)
    x2t = jnp.concatenate([xt, jnp.ones((1, q), jnp.float32)], axis=0)
    return pl.pallas_call(
        functools.partial(_argmax_body, k_total, nkb),
        grid=(nkb,),
        in_specs=[
            pl.BlockSpec((d + 1, q), lambda i: (0, 0)),
            pl.BlockSpec((d, _KB), lambda i: (0, i)),
        ],
        out_specs=pl.BlockSpec((q,), lambda i: (0,)),
        out_shape=jax.ShapeDtypeStruct((q,), jnp.int32),
        scratch_shapes=[
            pltpu.VMEM((q, 128), jnp.float32),
            pltpu.VMEM((q, 128), jnp.int32),
            pltpu.VMEM((d + 1, _KB), jnp.float32),
        ],
    )(x2t, inputs_t)


_BATCH = 8


def _gather_cols(targets_t, idx):
    t = targets_t.shape[0]
    q = idx.shape[0]
    b_per_w = q // _NUM_WORKERS
    q_per_core = q // _NUM_CORES
    mesh = plsc.VectorSubcoreMesh(core_axis_name="c", subcore_axis_name="s")

    @functools.partial(
        pl.kernel,
        out_type=jax.ShapeDtypeStruct((q, t), jnp.float32),
        mesh=mesh,
        compiler_params=pltpu.CompilerParams(needs_layout_passes=False),
        scratch_types=[
            pltpu.VMEM((b_per_w,), jnp.int32),
            pltpu.VMEM((_BATCH, t, 128), jnp.float32),
            pltpu.VMEM((b_per_w, t), jnp.float32),
            pltpu.SemaphoreType.DMA,
        ],
    )
    def gather(table_hbm, idx_hbm, out_hbm, idx_v, tile_v, rows_v, sem):
        cid = lax.axis_index("c")
        sid = lax.axis_index("s")
        base = (cid * _NUM_SUBCORES + sid) * b_per_w
        pltpu.sync_copy(idx_hbm.at[pl.ds(base, b_per_w)], idx_v)
        vecs = [idx_v[pl.ds(g * 16, 16)] for g in range(b_per_w // 16)]

        def col_of(qi):
            return vecs[qi // 16][qi % 16]

        for batch in range(b_per_w // _BATCH):
            cps = []
            for l in range(_BATCH):
                qi = batch * _BATCH + l
                colbase = pl.multiple_of((col_of(qi) >> 7) << 7, 128)
                cp = pltpu.make_async_copy(
                    table_hbm.at[:, pl.ds(colbase, 128)], tile_v.at[l], sem)
                cp.start()
                cps.append(cp)
            for cp in cps:
                cp.wait()
            for l in range(_BATCH):
                qi = batch * _BATCH + l
                col = col_of(qi)
                off = col - ((col >> 7) << 7)
                offv = jnp.zeros((16,), jnp.int32) + off
                for rb in range(t // 16):
                    rows = lax.iota(jnp.int32, 16) + rb * 16
                    vals = plsc.load_gather(tile_v.at[l], [rows, offv])
                    rows_v[qi, pl.ds(rb * 16, 16)] = vals

        pltpu.sync_copy(rows_v, out_hbm.at[pl.ds(base, b_per_w), :])

    return gather(targets_t, idx)


def kernel(x, inputs, targets):
    idx = _match_indices(x.T, inputs.T)
    return _gather_cols(targets.T, idx)


# speedup vs baseline: 1.0038x; 1.0038x over previous
import functools

import jax
import jax.numpy as jnp
from jax import lax
from jax.experimental import pallas as pl
from jax.experimental.pallas import tpu as pltpu
from jax.experimental.pallas import tpu_sc as plsc

_NUM_CORES = 2
_NUM_SUBCORES = 16
_NUM_WORKERS = _NUM_CORES * _NUM_SUBCORES

_KB = 4096
_ROWS = 64
_NEG = -1e30


def _argmax_body(k_total, nkb, xt_ref, kt_ref, out_ref, bm_ref, bb_ref):
    i = pl.program_id(0)
    q = bm_ref.shape[0]

    @pl.when(i == 0)
    def _():
        bm_ref[...] = jnp.full((q, 128), _NEG, jnp.float32)
        bb_ref[...] = jnp.zeros((q, 128), jnp.int32)

    kt = kt_ref[...]
    ksqh = 0.5 * jnp.sum(kt * kt, axis=0)
    s = lax.dot_general(
        xt_ref[...], kt, (((0,), (0,)), ((), ())),
        preferred_element_type=jnp.float32)
    sc = s - ksqh[None, :]

    valid_in_last = k_total - (nkb - 1) * _KB
    nslab = _KB // 128
    base0 = i * _KB

    def scan(scv):
        for qb in range(q // _ROWS):
            row = qb * _ROWS
            m = bm_ref[row:row + _ROWS, :]
            b = bb_ref[row:row + _ROWS, :]
            for j in range(nslab):
                v = scv[row:row + _ROWS, j * 128:(j + 1) * 128]
                upd = v > m
                b = jnp.where(upd, base0 + j * 128, b)
                m = jnp.where(upd, v, m)
            bm_ref[row:row + _ROWS, :] = m
            bb_ref[row:row + _ROWS, :] = b

    @pl.when(i < nkb - 1)
    def _():
        scan(sc)

    @pl.when(i == nkb - 1)
    def _():
        lane = lax.broadcasted_iota(jnp.int32, (q, _KB), 1)
        scan(jnp.where(lane < valid_in_last, sc, _NEG))

    @pl.when(i == nkb - 1)
    def _():
        m = bm_ref[...]
        b = bb_ref[...]
        g = jnp.max(m, axis=1)
        lane = lax.broadcasted_iota(jnp.int32, (q, 128), 1)
        cand = jnp.where(m == g[:, None], b + lane, jnp.int32(0x7FFFFFFF))
        out_ref[...] = jnp.min(cand, axis=1)


def _match_indices(xt, inputs_t):
    d, q = xt.shape
    k_total = inputs_t.shape[1]
    nkb = pl.cdiv(k_total, _KB)
    return pl.pallas_call(
        functools.partial(_argmax_body, k_total, nkb),
        grid=(nkb,),
        in_specs=[
            pl.BlockSpec((d, q), lambda i: (0, 0)),
            pl.BlockSpec((d, _KB), lambda i: (0, i)),
        ],
        out_specs=pl.BlockSpec((q,), lambda i: (0,)),
        out_shape=jax.ShapeDtypeStruct((q,), jnp.int32),
        scratch_shapes=[
            pltpu.VMEM((q, 128), jnp.float32),
            pltpu.VMEM((q, 128), jnp.int32),
        ],
    )(xt, inputs_t)


_BATCH = 8


def _gather_cols(targets_t, idx):
    t = targets_t.shape[0]
    q = idx.shape[0]
    b_per_w = q // _NUM_WORKERS
    q_per_core = q // _NUM_CORES
    mesh = plsc.VectorSubcoreMesh(core_axis_name="c", subcore_axis_name="s")

    @functools.partial(
        pl.kernel,
        out_type=jax.ShapeDtypeStruct((q, t), jnp.float32),
        mesh=mesh,
        compiler_params=pltpu.CompilerParams(needs_layout_passes=False),
        scratch_types=[
            pltpu.VMEM((b_per_w,), jnp.int32),
            pltpu.VMEM((_BATCH, t, 128), jnp.float32),
            pltpu.VMEM((b_per_w, t), jnp.float32),
            pltpu.SemaphoreType.DMA,
        ],
    )
    def gather(table_hbm, idx_hbm, out_hbm, idx_v, tile_v, rows_v, sem):
        cid = lax.axis_index("c")
        sid = lax.axis_index("s")
        base = (cid * _NUM_SUBCORES + sid) * b_per_w
        pltpu.sync_copy(idx_hbm.at[pl.ds(base, b_per_w)], idx_v)
        vecs = [idx_v[pl.ds(g * 16, 16)] for g in range(b_per_w // 16)]

        def col_of(qi):
            return vecs[qi // 16][qi % 16]

        for batch in range(b_per_w // _BATCH):
            cps = []
            for l in range(_BATCH):
                qi = batch * _BATCH + l
                colbase = pl.multiple_of((col_of(qi) >> 7) << 7, 128)
                cp = pltpu.make_async_copy(
                    table_hbm.at[:, pl.ds(colbase, 128)], tile_v.at[l], sem)
                cp.start()
                cps.append(cp)
            for cp in cps:
                cp.wait()
            for l in range(_BATCH):
                qi = batch * _BATCH + l
                col = col_of(qi)
                off = col - ((col >> 7) << 7)
                offv = jnp.zeros((16,), jnp.int32) + off
                for rb in range(t // 16):
                    rows = lax.iota(jnp.int32, 16) + rb * 16
                    vals = plsc.load_gather(tile_v.at[l], [rows, offv])
                    rows_v[qi, pl.ds(rb * 16, 16)] = vals

        pltpu.sync_copy(rows_v, out_hbm.at[pl.ds(base, b_per_w), :])

    return gather(targets_t, idx)


def kernel(x, inputs, targets):
    idx = _match_indices(x.T, inputs.T)
    return _gather_cols(targets.T, idx)
